# baseline (speedup 1.0000x reference)
import functools

import jax
import jax.numpy as jnp
from jax import lax
from jax.experimental import pallas as pl
from jax.experimental.pallas import tpu as pltpu
from jax.experimental.pallas import tpu_sc as plsc

_NUM_FIELDS = 26
_FIELD_DIM = 40000
_BATCH = 16384
_EMBED_DIM = 16
_TOTAL = _BATCH * _NUM_FIELDS
_NUM_WORKERS = 32
_ITEMS = _NUM_FIELDS * (_BATCH // 128)
_ITEMS_PER_W = _ITEMS // _NUM_WORKERS
_XPW = _ITEMS_PER_W * 128
_TR_STRIDE = 8125 * 1024
_TC_STRIDE = 1024


def _sc_embedding_lookup(x_flat, t_flat):
    mesh = plsc.VectorSubcoreMesh(core_axis_name="c", subcore_axis_name="s")

    @functools.partial(
        pl.kernel,
        mesh=mesh,
        compiler_params=pltpu.CompilerParams(
            use_tc_tiling_on_sc=False, needs_layout_passes=False),
        out_type=jax.ShapeDtypeStruct((26, 2, 128, 1024), jnp.float32),
        scratch_types=[
            pltpu.VMEM((_XPW,), jnp.int32),
            pltpu.VMEM((4 * 2048,), jnp.int32),
            pltpu.VMEM((4 * 2048,), jnp.float32),
            pltpu.VMEM((4 * 2048,), jnp.float32),
            pltpu.SemaphoreType.DMA,
            pltpu.SemaphoreType.DMA,
            pltpu.SemaphoreType.DMA,
        ],
    )
    def k(x_hbm, t_hbm, out_hbm, xb, idxb, gb, tb, xsem, gsem, osem):
        wid = lax.axis_index("s") * 2 + lax.axis_index("c")
        t0 = wid * _ITEMS_PER_W
        pltpu.async_copy(
            x_hbm.at[pl.ds(t0 * 128, _XPW)], xb, xsem).wait()

        def drain(sem, words):
            pltpu.make_async_copy(
                t_hbm.at[pl.ds(0, words)], gb.at[pl.ds(0, words)], sem
            ).wait()

        def finish_item(i):
            t = t0 + i
            f = lax.shift_right_logical(t, 7)
            bt = lax.bitwise_and(t, 127)
            q = lax.bitwise_and(i, 3) * 2048
            iot = lax.iota(jnp.int32, 16) * 16 + q
            for k in range(16):
                for j in range(8):
                    pos = iot + (j * 256 + k)
                    tb[pl.ds(q + k * 128 + j * 16, 16)] = plsc.load_gather(
                        gb, [pos])
            for dt in range(2):
                pltpu.async_copy(
                    tb.at[pl.ds(q + dt * 1024, 1024)],
                    out_hbm.at[f, dt, bt], osem)

        @pl.loop(0, _ITEMS_PER_W)
        def item_loop(i):
            t = t0 + i
            f = lax.shift_right_logical(t, 7)
            foff = f * _FIELD_DIM
            q = lax.bitwise_and(i, 3) * 2048
            iot = lax.iota(jnp.int32, 16) * 16 + q
            for j in range(8):
                xv = xb[pl.ds(i * 128 + j * 16, 16)]
                r = xv + foff
                base = lax.shift_left(
                    lax.shift_right_logical(r, 7), 10
                ) + lax.bitwise_and(r, 127)
                for k in range(16):
                    pos = iot + (j * 256 + k)
                    plsc.store_scatter(
                        idxb, [pos],
                        base + ((k // 8) * _TR_STRIDE + (k % 8) * 128))

            @pl.when(i >= 3)
            def _():
                drain(gsem, 2048)

            @pl.when(i >= 7)
            def _():
                drain(osem, 2048)

            for h in range(4):
                o = q + h * 512
                pltpu.async_copy(
                    t_hbm.at[idxb.at[pl.ds(o, 512)]],
                    gb.at[pl.ds(o, 512)], gsem)

            @pl.when(i >= 3)
            def _():
                finish_item(i - 3)

        for e in range(3):
            drain(gsem, 2048)
            finish_item(_ITEMS_PER_W - 3 + e)
        drain(osem, 7 * 2048)

    return k(x_flat, t_flat)


def kernel(x, table):
    x_flat = jnp.transpose(x.astype(jnp.int32)).reshape(_TOTAL)
    t_flat = (
        jnp.transpose(table)
        .reshape(2, 8, 8125, 128)
        .transpose(0, 2, 1, 3)
        .reshape(2 * 8125 * 8 * 128)
    )
    out4 = _sc_embedding_lookup(x_flat, t_flat)
    out5 = out4.reshape(_NUM_FIELDS, 2, 128, 8, 128)
    return (
        out5.transpose(0, 1, 3, 2, 4)
        .reshape(_NUM_FIELDS, _EMBED_DIM, _BATCH)
        .transpose(2, 0, 1)
    )


# speedup vs baseline: 1.1099x; 1.1099x over previous
import functools

import jax
import jax.numpy as jnp
from jax import lax
from jax.experimental import pallas as pl
from jax.experimental.pallas import tpu as pltpu
from jax.experimental.pallas import tpu_sc as plsc

_NUM_FIELDS = 26
_FIELD_DIM = 40000
_BATCH = 16384
_EMBED_DIM = 16
_TOTAL = _BATCH * _NUM_FIELDS
_NUM_WORKERS = 32
_ITEMS = _NUM_FIELDS * (_BATCH // 128)
_ITEMS_PER_W = _ITEMS // _NUM_WORKERS
_XPW = _ITEMS_PER_W * 128
_TR_STRIDE = 8125 * 1024
_TC_STRIDE = 1024


def _sc_embedding_lookup(x_flat, t_flat):
    mesh = plsc.VectorSubcoreMesh(core_axis_name="c", subcore_axis_name="s")

    @functools.partial(
        pl.kernel,
        mesh=mesh,
        compiler_params=pltpu.CompilerParams(use_tc_tiling_on_sc=False),
        out_type=jax.ShapeDtypeStruct((26, 2, 128, 1024), jnp.float32),
        scratch_types=[
            pltpu.VMEM((_XPW,), jnp.int32),
            pltpu.VMEM((8 * 2048,), jnp.int32),
            pltpu.VMEM((8 * 2048,), jnp.float32),
            pltpu.SemaphoreType.DMA,
            pltpu.SemaphoreType.DMA,
            pltpu.SemaphoreType.DMA,
        ],
    )
    def k(x_hbm, t_hbm, out_hbm, xb, idxb, gb, xsem, gsem, osem):
        wid = lax.axis_index("s") * 2 + lax.axis_index("c")
        t0 = wid * _ITEMS_PER_W
        pltpu.async_copy(
            x_hbm.at[pl.ds(t0 * 128, _XPW)], xb, xsem).wait()

        def drain(sem, words):
            pltpu.make_async_copy(
                t_hbm.at[pl.ds(0, words)], gb.at[pl.ds(0, words)], sem
            ).wait()

        def fire_out(i):
            t = t0 + i
            f = lax.shift_right_logical(t, 7)
            bt = lax.bitwise_and(t, 127)
            q = lax.bitwise_and(i, 7) * 2048
            for dt in range(2):
                pltpu.async_copy(
                    gb.at[pl.ds(q + dt * 1024, 1024)],
                    out_hbm.at[f, dt, bt], osem)

        @pl.loop(0, _ITEMS_PER_W)
        def item_loop(i):
            t = t0 + i
            f = lax.shift_right_logical(t, 7)
            foff = f * _FIELD_DIM
            q = lax.bitwise_and(i, 7) * 2048
            for j in range(8):
                xv = xb[pl.ds(i * 128 + j * 16, 16)]
                r = xv + foff
                base = lax.shift_left(
                    lax.shift_right_logical(r, 7), 10
                ) + lax.bitwise_and(r, 127)
                for dt in range(2):
                    for s in range(8):
                        idxb[pl.ds(q + dt * 1024 + s * 128 + j * 16, 16)] = (
                            base + (dt * _TR_STRIDE + s * 128))

            @pl.when(i >= 5)
            def _():
                drain(gsem, 2048)

            @pl.when(i >= 8)
            def _():
                drain(osem, 2048)

            for dt in range(2):
                for h in range(2):
                    o = q + dt * 1024 + h * 512
                    pltpu.async_copy(
                        t_hbm.at[idxb.at[pl.ds(o, 512)]],
                        gb.at[pl.ds(o, 512)], gsem)

            @pl.when(i >= 5)
            def _():
                fire_out(i - 5)

        for e in range(5):
            drain(gsem, 2048)
            fire_out(_ITEMS_PER_W - 5 + e)
        drain(osem, 8 * 2048)

    return k(x_flat, t_flat)


def kernel(x, table):
    x_flat = jnp.transpose(x.astype(jnp.int32)).reshape(_TOTAL)
    t_flat = (
        jnp.transpose(table)
        .reshape(2, 8, 8125, 128)
        .transpose(0, 2, 1, 3)
        .reshape(2 * 8125 * 8 * 128)
    )
    out4 = _sc_embedding_lookup(x_flat, t_flat)
    out5 = out4.reshape(_NUM_FIELDS, 2, 128, 8, 128)
    return (
        out5.transpose(0, 1, 3, 2, 4)
        .reshape(_NUM_FIELDS, _EMBED_DIM, _BATCH)
        .transpose(2, 0, 1)
    )


# speedup vs baseline: 1.1539x; 1.0397x over previous
import functools

import jax
import jax.numpy as jnp
from jax import lax
from jax.experimental import pallas as pl
from jax.experimental.pallas import tpu as pltpu
from jax.experimental.pallas import tpu_sc as plsc

_NUM_FIELDS = 26
_FIELD_DIM = 40000
_BATCH = 16384
_EMBED_DIM = 16
_TOTAL = _BATCH * _NUM_FIELDS
_NUM_WORKERS = 32
_ITEMS = _NUM_FIELDS * (_BATCH // 128)
_ITEMS_PER_W = _ITEMS // _NUM_WORKERS
_XPW = _ITEMS_PER_W * 128
_TR_STRIDE = 8125 * 1024
_TC_STRIDE = 1024


def _sc_embedding_lookup(x_flat, t_flat):
    mesh = plsc.VectorSubcoreMesh(core_axis_name="c", subcore_axis_name="s")

    @functools.partial(
        pl.kernel,
        mesh=mesh,
        compiler_params=pltpu.CompilerParams(use_tc_tiling_on_sc=False),
        out_type=jax.ShapeDtypeStruct((26, 2, 128, 1024), jnp.float32),
        scratch_types=[
            pltpu.VMEM((_XPW,), jnp.int32),
            pltpu.VMEM((8 * 2048,), jnp.int32),
            pltpu.VMEM((8 * 2048,), jnp.float32),
            pltpu.SemaphoreType.DMA,
            pltpu.SemaphoreType.DMA,
            pltpu.SemaphoreType.DMA,
        ],
    )
    def k(x_hbm, t_hbm, out_hbm, xb, idxb, gb, xsem, gsem, osem):
        wid = lax.axis_index("s") * 2 + lax.axis_index("c")
        t0 = wid * _ITEMS_PER_W
        pltpu.async_copy(
            x_hbm.at[pl.ds(t0 * 128, _XPW)], xb, xsem).wait()

        def drain(sem, words):
            pltpu.make_async_copy(
                t_hbm.at[pl.ds(0, words)], gb.at[pl.ds(0, words)], sem
            ).wait()

        def fire_out(i):
            t = t0 + i
            f = lax.shift_right_logical(t, 7)
            bt = lax.bitwise_and(t, 127)
            q = lax.bitwise_and(i, 7) * 2048
            for dt in range(2):
                pltpu.async_copy(
                    gb.at[pl.ds(q + dt * 1024, 1024)],
                    out_hbm.at[f, dt, bt], osem)

        @pl.loop(0, _ITEMS_PER_W)
        def item_loop(i):
            t = t0 + i
            f = lax.shift_right_logical(t, 7)
            foff = f * _FIELD_DIM
            q = lax.bitwise_and(i, 7) * 2048
            for j in range(8):
                xv = xb[pl.ds(i * 128 + j * 16, 16)]
                r = xv + foff
                base = lax.shift_left(
                    lax.shift_right_logical(r, 7), 10
                ) + lax.bitwise_and(r, 127)
                for dt in range(2):
                    for s in range(8):
                        idxb[pl.ds(q + dt * 1024 + s * 128 + j * 16, 16)] = (
                            base + (dt * _TR_STRIDE + s * 128))

            @pl.when(i >= 7)
            def _():
                drain(gsem, 2048)

            @pl.when(i >= 8)
            def _():
                drain(osem, 2048)

            for dt in range(2):
                for h in range(2):
                    o = q + dt * 1024 + h * 512
                    pltpu.async_copy(
                        t_hbm.at[idxb.at[pl.ds(o, 512)]],
                        gb.at[pl.ds(o, 512)], gsem)

            @pl.when(i >= 7)
            def _():
                fire_out(i - 7)

        for e in range(7):
            drain(gsem, 2048)
            fire_out(_ITEMS_PER_W - 7 + e)
        drain(osem, 8 * 2048)

    return k(x_flat, t_flat)


def kernel(x, table):
    x_flat = jnp.transpose(x.astype(jnp.int32)).reshape(_TOTAL)
    t_flat = (
        jnp.transpose(table)
        .reshape(2, 8, 8125, 128)
        .transpose(0, 2, 1, 3)
        .reshape(2 * 8125 * 8 * 128)
    )
    out4 = _sc_embedding_lookup(x_flat, t_flat)
    out5 = out4.reshape(_NUM_FIELDS, 2, 128, 8, 128)
    return (
        out5.transpose(0, 1, 3, 2, 4)
        .reshape(_NUM_FIELDS, _EMBED_DIM, _BATCH)
        .transpose(2, 0, 1)
    )


# speedup vs baseline: 1.2349x; 1.0701x over previous
import functools

import jax
import jax.numpy as jnp
from jax import lax
from jax.experimental import pallas as pl
from jax.experimental.pallas import tpu as pltpu
from jax.experimental.pallas import tpu_sc as plsc

_NUM_FIELDS = 26
_FIELD_DIM = 40000
_BATCH = 16384
_EMBED_DIM = 16
_TOTAL = _BATCH * _NUM_FIELDS
_NUM_WORKERS = 32
_ITEMS = _NUM_FIELDS * (_BATCH // 128)
_ITEMS_PER_W = _ITEMS // _NUM_WORKERS
_XPW = _ITEMS_PER_W * 128
_TR_STRIDE = 8125 * 1024
_TC_STRIDE = 1024


def _sc_embedding_lookup(x_flat, t_flat):
    mesh = plsc.VectorSubcoreMesh(core_axis_name="c", subcore_axis_name="s")

    @functools.partial(
        pl.kernel,
        mesh=mesh,
        compiler_params=pltpu.CompilerParams(use_tc_tiling_on_sc=False),
        out_type=jax.ShapeDtypeStruct((26, 2, 128, 1024), jnp.float32),
        scratch_types=[
            pltpu.VMEM((_XPW,), jnp.int32),
            pltpu.VMEM((16 * 2048,), jnp.int32),
            pltpu.VMEM((16 * 2048,), jnp.float32),
            pltpu.SemaphoreType.DMA,
            pltpu.SemaphoreType.DMA,
            pltpu.SemaphoreType.DMA,
        ],
    )
    def k(x_hbm, t_hbm, out_hbm, xb, idxb, gb, xsem, gsem, osem):
        wid = lax.axis_index("s") * 2 + lax.axis_index("c")
        t0 = wid * _ITEMS_PER_W
        pltpu.async_copy(
            x_hbm.at[pl.ds(t0 * 128, _XPW)], xb, xsem).wait()

        def drain(sem, words):
            pltpu.make_async_copy(
                t_hbm.at[pl.ds(0, words)], gb.at[pl.ds(0, words)], sem
            ).wait()

        def fire_out(i):
            t = t0 + i
            f = lax.shift_right_logical(t, 7)
            bt = lax.bitwise_and(t, 127)
            q = lax.bitwise_and(i, 15) * 2048
            for dt in range(2):
                pltpu.async_copy(
                    gb.at[pl.ds(q + dt * 1024, 1024)],
                    out_hbm.at[f, dt, bt], osem)

        @pl.loop(0, _ITEMS_PER_W)
        def item_loop(i):
            t = t0 + i
            f = lax.shift_right_logical(t, 7)
            foff = f * _FIELD_DIM
            q = lax.bitwise_and(i, 15) * 2048
            for j in range(8):
                xv = xb[pl.ds(i * 128 + j * 16, 16)]
                r = xv + foff
                base = lax.shift_left(
                    lax.shift_right_logical(r, 7), 10
                ) + lax.bitwise_and(r, 127)
                for dt in range(2):
                    for s in range(8):
                        idxb[pl.ds(q + dt * 1024 + s * 128 + j * 16, 16)] = (
                            base + (dt * _TR_STRIDE + s * 128))

            @pl.when(i >= 12)
            def _():
                drain(gsem, 2048)

            @pl.when(i >= 16)
            def _():
                drain(osem, 2048)

            for dt in range(2):
                for h in range(2):
                    o = q + dt * 1024 + h * 512
                    pltpu.async_copy(
                        t_hbm.at[idxb.at[pl.ds(o, 512)]],
                        gb.at[pl.ds(o, 512)], gsem)

            @pl.when(i >= 12)
            def _():
                fire_out(i - 12)

        for e in range(12):
            drain(gsem, 2048)
            fire_out(_ITEMS_PER_W - 12 + e)
        drain(osem, 16 * 2048)

    return k(x_flat, t_flat)


def kernel(x, table):
    x_flat = jnp.transpose(x.astype(jnp.int32)).reshape(_TOTAL)
    t_flat = (
        jnp.transpose(table)
        .reshape(2, 8, 8125, 128)
        .transpose(0, 2, 1, 3)
        .reshape(2 * 8125 * 8 * 128)
    )
    out4 = _sc_embedding_lookup(x_flat, t_flat)
    out5 = out4.reshape(_NUM_FIELDS, 2, 128, 8, 128)
    return (
        out5.transpose(0, 1, 3, 2, 4)
        .reshape(_NUM_FIELDS, _EMBED_DIM, _BATCH)
        .transpose(2, 0, 1)
    )
